# Initial kernel scaffold; baseline (speedup 1.0000x reference)
import jax
import jax.numpy as jnp
from jax.experimental import pallas as pl


def kernel(h, edge_index, e, W_emb, b_emb, Ws, bs, gammas, betas):
    raise NotImplementedError("write your pallas kernel here")


# speedup vs baseline: 5.0273x; 5.0273x over previous
import functools

import jax
import jax.numpy as jnp
from jax import lax
from jax.experimental import pallas as pl
from jax.experimental.pallas import tpu as pltpu
from jax.experimental.pallas import tpu_sc as plsc

_N = 10000
_E = 320000
_D = 128
_NC = 2
_NS = 16
_NW = _NC * _NS
_CHUNK = 128
_NCHUNKS = _E // _CHUNK
_CPW = -(-_NCHUNKS // _NW)
_NP = 10240
_RPT = _NP // _NS
_DEGW = 16


def _sc_mesh():
    return plsc.VectorSubcoreMesh(
        core_axis_name="c", subcore_axis_name="s",
        num_cores=_NC, num_subcores=_NS)


_DCP = -(-_NCHUNKS // _NS)


def _degree_partials(src, dst, zeros_nd, ones_nd):
    @functools.partial(
        pl.kernel,
        out_type=jax.ShapeDtypeStruct((_NC, _NP, _D), jnp.float32),
        mesh=_sc_mesh(),
        scratch_types=[
            pltpu.VMEM((_CHUNK,), jnp.int32),
            pltpu.VMEM((_CHUNK, _D), jnp.float32),
            pltpu.VMEM_SHARED((_NP, _D), jnp.float32),
        ],
    )
    def deg_k(src_hbm, dst_hbm, z_hbm, o_hbm, out_hbm, idx, ones_v, acc_sh):
        cid = lax.axis_index("c")
        sid = lax.axis_index("s")
        r0 = sid * _RPT
        pltpu.sync_copy(z_hbm.at[pl.ds(r0, _RPT)], acc_sh.at[pl.ds(r0, _RPT)])
        pltpu.sync_copy(o_hbm, ones_v)
        plsc.subcore_barrier()

        def body(j, carry):
            chunk = j * _NS + sid

            @pl.when(chunk < _NCHUNKS)
            def _():
                base = chunk * _CHUNK

                @pl.when(cid == 0)
                def _():
                    pltpu.sync_copy(src_hbm.at[pl.ds(base, _CHUNK)], idx)

                @pl.when(cid == 1)
                def _():
                    pltpu.sync_copy(dst_hbm.at[pl.ds(base, _CHUNK)], idx)

                pltpu.sync_copy(ones_v, acc_sh.at[idx], add=True)

            return carry

        lax.fori_loop(0, _DCP, body, 0)
        plsc.subcore_barrier()
        pltpu.sync_copy(acc_sh.at[pl.ds(r0, _RPT)],
                        out_hbm.at[cid, pl.ds(r0, _RPT)])

    return deg_k(src, dst, zeros_nd, ones_nd)


def _aggregate_partials(xs, src, dst, zeros_nd):
    @functools.partial(
        pl.kernel,
        out_type=jax.ShapeDtypeStruct((_NC, _NP, _D), jnp.float32),
        mesh=_sc_mesh(),
        scratch_types=[
            pltpu.VMEM((_CHUNK,), jnp.int32),
            pltpu.VMEM((_CHUNK,), jnp.int32),
            pltpu.VMEM((_CHUNK, _D), jnp.float32),
            pltpu.VMEM_SHARED((_NP, _D), jnp.float32),
        ],
    )
    def agg_k(xs_hbm, src_hbm, dst_hbm, z_hbm, out_hbm,
              sidx, didx, rows, agg_sh):
        cid = lax.axis_index("c")
        sid = lax.axis_index("s")
        wid = cid * _NS + sid
        r0 = sid * _RPT
        pltpu.sync_copy(z_hbm.at[pl.ds(r0, _RPT)], agg_sh.at[pl.ds(r0, _RPT)])
        plsc.subcore_barrier()

        def body(j, carry):
            chunk = j * _NW + wid

            @pl.when(chunk < _NCHUNKS)
            def _():
                base = chunk * _CHUNK
                pltpu.sync_copy(src_hbm.at[pl.ds(base, _CHUNK)], sidx)
                pltpu.sync_copy(dst_hbm.at[pl.ds(base, _CHUNK)], didx)
                pltpu.sync_copy(xs_hbm.at[sidx], rows)
                pltpu.sync_copy(rows, agg_sh.at[didx], add=True)

            return carry

        lax.fori_loop(0, _CPW, body, 0)
        plsc.subcore_barrier()
        pltpu.sync_copy(agg_sh.at[pl.ds(r0, _RPT)],
                        out_hbm.at[cid, pl.ds(r0, _RPT)])

    return agg_k(xs, src, dst, zeros_nd)


def _emb_body(h_ref, w_ref, b_ref, degp_ref, x_ref, xs_ref, cin_ref, cout_ref):
    x = jnp.dot(h_ref[...], w_ref[...],
                preferred_element_type=jnp.float32) + b_ref[...]
    deg_out = degp_ref[0, 0:_N, 0:1]
    deg_in = degp_ref[1, 0:_N, 0:1]
    c_out = lax.rsqrt(jnp.maximum(deg_out, 1.0))
    c_in = lax.rsqrt(jnp.maximum(deg_in, 1.0))
    x_ref[...] = x
    xs_ref[...] = x * c_out
    cin_ref[...] = c_in
    cout_ref[...] = c_out


def _embed(h, W_emb, b_emb, degp):
    return pl.pallas_call(
        _emb_body,
        out_shape=[
            jax.ShapeDtypeStruct((_N, _D), jnp.float32),
            jax.ShapeDtypeStruct((_N, _D), jnp.float32),
            jax.ShapeDtypeStruct((_N, 1), jnp.float32),
            jax.ShapeDtypeStruct((_N, 1), jnp.float32),
        ],
    )(h, W_emb, b_emb.reshape(1, _D), degp)


def _layer_body(p_ref, cin_ref, cout_ref, w_ref, b_ref, g_ref, bt_ref,
                xin_ref, xout_ref, xsout_ref):
    agg = (p_ref[0, 0:_N] + p_ref[1, 0:_N]) * cin_ref[...]
    y = jnp.dot(agg, w_ref[...],
                preferred_element_type=jnp.float32) + b_ref[...]
    mu = jnp.mean(y, axis=0, keepdims=True)
    yc = y - mu
    var = jnp.mean(yc * yc, axis=0, keepdims=True)
    yn = yc * lax.rsqrt(var + 1e-5) * g_ref[...] + bt_ref[...]
    x_new = xin_ref[...] + jnp.maximum(yn, 0.0)
    xout_ref[...] = x_new
    xsout_ref[...] = x_new * cout_ref[...]


def _layer(parts, c_in, c_out, W, b, g, bt, x_in):
    return pl.pallas_call(
        _layer_body,
        out_shape=[
            jax.ShapeDtypeStruct((_N, _D), jnp.float32),
            jax.ShapeDtypeStruct((_N, _D), jnp.float32),
        ],
    )(parts, c_in, c_out, W, b.reshape(1, _D), g.reshape(1, _D),
      bt.reshape(1, _D), x_in)


def kernel(h, edge_index, e, W_emb, b_emb, Ws, bs, gammas, betas):
    del e
    src = edge_index[0]
    dst = edge_index[1]
    zeros_nd = jnp.zeros((_NP, _D), jnp.float32)
    ones_nd = jnp.ones((_CHUNK, _D), jnp.float32)

    degp = _degree_partials(src, dst, zeros_nd, ones_nd)
    x, xs, c_in, c_out = _embed(h, W_emb, b_emb, degp)
    for l in range(4):
        parts = _aggregate_partials(xs, src, dst, zeros_nd)
        x, xs = _layer(parts, c_in, c_out, Ws[l], bs[l], gammas[l],
                       betas[l], x)
    return x
